# Initial kernel scaffold; baseline (speedup 1.0000x reference)
import jax
import jax.numpy as jnp
from jax.experimental import pallas as pl


def kernel(src, params):
    raise NotImplementedError("write your pallas kernel here")


# speedup vs baseline: 4.0994x; 4.0994x over previous
import functools

import jax
import jax.numpy as jnp
from jax import lax
from jax.experimental import pallas as pl
from jax.experimental.pallas import tpu as pltpu

D = 1024
H = 16
DH = 64
DFF = 4096
N = 2048
BC = 4
SC = 2
BS = 4
NSEL = 2
RT = 256
NRT = N // RT
NC2 = N // SC
SCALE = DH ** -0.5

_call = pl.pallas_call


def _s1_body(src_r, wq_r, wk_r, wv_r, wg_r, g1_r, b1_r, bg_r,
             q_r, k_r, v_r, g_r):
    x = src_r[...]
    m = jnp.mean(x, -1, keepdims=True)
    va = jnp.mean((x - m) ** 2, -1, keepdims=True)
    xln = (x - m) / jnp.sqrt(va + 1e-5) * g1_r[...] + b1_r[...]
    q_r[...] = xln @ wq_r[...]
    k_r[...] = xln @ wk_r[...]
    v_r[...] = xln @ wv_r[...]
    g_r[...] = xln @ wg_r[...] + bg_r[...]


def _s1(src, p):
    full = lambda a, b: pl.BlockSpec((a, b), lambda i: (0, 0))
    return _call(
        _s1_body,
        grid=(NRT,),
        in_specs=[
            pl.BlockSpec((RT, D), lambda i: (i, 0)),
            full(D, H * DH), full(D, H * DH), full(D, H * DH), full(D, H * 3),
            full(1, D), full(1, D), full(1, H * 3),
        ],
        out_specs=[
            pl.BlockSpec((RT, H * DH), lambda i: (i, 0)),
            pl.BlockSpec((RT, H * DH), lambda i: (i, 0)),
            pl.BlockSpec((RT, H * DH), lambda i: (i, 0)),
            pl.BlockSpec((RT, H * 3), lambda i: (i, 0)),
        ],
        out_shape=[
            jax.ShapeDtypeStruct((N, H * DH), jnp.float32),
            jax.ShapeDtypeStruct((N, H * DH), jnp.float32),
            jax.ShapeDtypeStruct((N, H * DH), jnp.float32),
            jax.ShapeDtypeStruct((N, H * 3), jnp.float32),
        ],
    )(src, p['Wq'], p['Wk'], p['Wv'], p['Wg'],
      p['ln1_g'].reshape(1, D), p['ln1_b'].reshape(1, D),
      p['bg'].reshape(1, H * 3))


def _s2_body(kev_r, kod_r, vev_r, vod_r, wkc_r, bkc_r, wvc_r, bvc_r,
             ck_r, cv_r):
    kev = kev_r[0]
    kod = kod_r[0]
    vev = vev_r[0]
    vod = vod_r[0]
    z = jnp.zeros((1, DH), jnp.float32)
    kev1 = jnp.concatenate([kev[1:], z], axis=0)
    kod1 = jnp.concatenate([kod[1:], z], axis=0)
    vev1 = jnp.concatenate([vev[1:], z], axis=0)
    vod1 = jnp.concatenate([vod[1:], z], axis=0)
    ckc = jnp.concatenate([kev, kod, kev1, kod1], axis=1)
    cvc = jnp.concatenate([vev, vod, vev1, vod1], axis=1)
    ck_r[0] = ckc @ wkc_r[...] + bkc_r[...]
    cv_r[0] = cvc @ wvc_r[...] + bvc_r[...]


def _s2(kev, kod, vev, vod, p):
    blk = pl.BlockSpec((1, NC2, DH), lambda h: (h, 0, 0))
    full = lambda a, b: pl.BlockSpec((a, b), lambda h: (0, 0))
    return _call(
        _s2_body,
        grid=(H,),
        in_specs=[blk, blk, blk, blk,
                  full(BC * DH, DH), full(1, DH), full(BC * DH, DH), full(1, DH)],
        out_specs=[blk, blk],
        out_shape=[jax.ShapeDtypeStruct((H, NC2, DH), jnp.float32),
                   jax.ShapeDtypeStruct((H, NC2, DH), jnp.float32)],
    )(kev, kod, vev, vod, p['Wkc'], p['bkc'].reshape(1, DH),
      p['Wvc'], p['bvc'].reshape(1, DH))


def _s3_body(q_r, k_r, v_r, kp_r, vp_r, ck_r, cv_r,
             cout_r, wout_r, tok_r):
    h = pl.program_id(0)
    rt = pl.program_id(1)
    q = q_r[0]
    ck = ck_r[0]
    cv = cv_r[0]
    irow = rt * RT + lax.broadcasted_iota(jnp.int32, (RT, NC2), 0)
    col = lax.broadcasted_iota(jnp.int32, (RT, NC2), 1)

    clog = lax.dot_general(q, ck, (((1,), (1,)), ((), ()))) * SCALE
    cmask = (2 * col + BC - 1) <= irow
    clogm = jnp.where(cmask, clog, -1e9)
    rowmax = jnp.max(clogm, -1, keepdims=True)
    e = jnp.where(cmask, jnp.exp(clogm - rowmax), 0.0)
    den = jnp.sum(e, -1, keepdims=True)
    cattn = e / jnp.where(den == 0.0, 1.0, den)
    cout_r[0] = lax.dot_general(cattn, cv, (((1,), (0,)), ((), ())))

    esh = jnp.concatenate([e[:, 1:], jnp.zeros((RT, 1), jnp.float32)], axis=1)
    pair = e + esh
    scores = jnp.where((col % 2 == 0) & (2 * col <= irow), pair, -1.0)
    m1 = jnp.max(scores, -1, keepdims=True)
    i1 = jnp.min(jnp.where(scores == m1, col, NC2 * 4), -1, keepdims=True)
    scores2 = jnp.where(col == i1, -1.0, scores)
    m2 = jnp.max(scores2, -1, keepdims=True)
    i2 = jnp.min(jnp.where(scores2 == m2, col, NC2 * 4), -1, keepdims=True)
    sel1 = i1 // 2
    sel2 = i2 // 2
    c8 = lax.broadcasted_iota(jnp.int32, (RT, NSEL * BS), 1)
    blkid = jnp.where(c8 < BS, sel1, sel2)
    tok_r[0] = blkid * BS + (c8 % BS) + h * N

    k = k_r[0]
    v = v_r[0]
    kp = kp_r[0]
    vp = vp_r[0]
    ipc = rt * RT + lax.broadcasted_iota(jnp.int32, (RT, 1), 0)
    d1 = jnp.sum(q * k, -1, keepdims=True) * SCALE
    d0 = jnp.sum(q * kp, -1, keepdims=True) * SCALE
    valid0 = ipc >= 1
    d0m = jnp.where(valid0, d0, -1e9)
    mw = jnp.maximum(d0m, d1)
    e0 = jnp.where(valid0, jnp.exp(d0m - mw), 0.0)
    e1 = jnp.exp(d1 - mw)
    wout_r[0] = (e0 * vp + e1 * v) / (e0 + e1)


def _s3(q3, k3, v3, kp3, vp3, ck3, cv3):
    rblk = pl.BlockSpec((1, RT, DH), lambda h, r: (h, r, 0))
    cblk = pl.BlockSpec((1, NC2, DH), lambda h, r: (h, 0, 0))
    return _call(
        _s3_body,
        grid=(H, NRT),
        in_specs=[rblk, rblk, rblk, rblk, rblk, cblk, cblk],
        out_specs=[rblk, rblk,
                   pl.BlockSpec((1, RT, NSEL * BS), lambda h, r: (h, r, 0))],
        out_shape=[jax.ShapeDtypeStruct((H, N, DH), jnp.float32),
                   jax.ShapeDtypeStruct((H, N, DH), jnp.float32),
                   jax.ShapeDtypeStruct((H, N, NSEL * BS), jnp.int32)],
    )(q3, k3, v3, kp3, vp3, ck3, cv3)


def _s35_body(q_r, tok_r, ks_r, vs_r, sout_r):
    h = pl.program_id(0)
    rt = pl.program_id(1)
    q = q_r[0]
    ks = ks_r[0].reshape(RT, NSEL * BS, DH)
    vs = vs_r[0].reshape(RT, NSEL * BS, DH)
    tok = tok_r[0] - h * N
    ipc = rt * RT + lax.broadcasted_iota(jnp.int32, (RT, 1), 0)
    logs = []
    for j in range(NSEL * BS):
        lj = jnp.sum(q * ks[:, j, :], -1, keepdims=True) * SCALE
        tm = tok[:, j:j + 1] <= ipc
        logs.append(jnp.where(tm, lj, -1e9))
    m = functools.reduce(jnp.maximum, logs)
    es = [jnp.exp(l - m) for l in logs]
    den = functools.reduce(jnp.add, es)
    acc = es[0] * vs[:, 0, :]
    for j in range(1, NSEL * BS):
        acc = acc + es[j] * vs[:, j, :]
    sout_r[0] = acc / den


def _s35(q3, tok, ksel, vsel):
    rblk = pl.BlockSpec((1, RT, DH), lambda h, r: (h, r, 0))
    sblk = pl.BlockSpec((1, RT * NSEL * BS, DH), lambda h, r: (h, r, 0))
    return _call(
        _s35_body,
        grid=(H, NRT),
        in_specs=[rblk,
                  pl.BlockSpec((1, RT, NSEL * BS), lambda h, r: (h, r, 0)),
                  sblk, sblk],
        out_specs=[rblk],
        out_shape=[jax.ShapeDtypeStruct((H, N, DH), jnp.float32)],
    )(q3, tok, ksel, vsel)[0]


def _s4a_body(cout_r, sout_r, wout_r, g_r, wo_r, src_r, bo_r, h1_r):
    h = pl.program_id(1)
    g = jax.nn.sigmoid(g_r[0])
    o_h = (g[:, 0:1] * cout_r[0] + g[:, 1:2] * sout_r[0]
           + g[:, 2:3] * wout_r[0])
    part = o_h @ wo_r[0]

    @pl.when(h == 0)
    def _():
        h1_r[...] = src_r[...] + bo_r[...] + part

    @pl.when(h != 0)
    def _():
        h1_r[...] += part


def _s4a(cout3, sout3, wout3, g3, wo3, src, bo):
    rblk = pl.BlockSpec((1, RT, DH), lambda r, h: (h, r, 0))
    return _call(
        _s4a_body,
        grid=(NRT, H),
        in_specs=[rblk, rblk, rblk,
                  pl.BlockSpec((1, RT, 3), lambda r, h: (h, r, 0)),
                  pl.BlockSpec((1, DH, D), lambda r, h: (h, 0, 0)),
                  pl.BlockSpec((RT, D), lambda r, h: (r, 0)),
                  pl.BlockSpec((1, D), lambda r, h: (0, 0))],
        out_specs=[pl.BlockSpec((RT, D), lambda r, h: (r, 0))],
        out_shape=[jax.ShapeDtypeStruct((N, D), jnp.float32)],
    )(cout3, sout3, wout3, g3, wo3, src, bo.reshape(1, D))[0]


def _s4b_body(h1_r, g2_r, b2ln_r, w1_r, b1_r, w2_r, b2_r, out_r, y_scr):
    j = pl.program_id(1)

    @pl.when(j == 0)
    def _():
        x = h1_r[...]
        m = jnp.mean(x, -1, keepdims=True)
        va = jnp.mean((x - m) ** 2, -1, keepdims=True)
        y_scr[...] = (x - m) / jnp.sqrt(va + 1e-5) * g2_r[...] + b2ln_r[...]
        out_r[...] = x + b2_r[...]

    hmid = jax.nn.gelu(y_scr[...] @ w1_r[...] + b1_r[...])
    out_r[...] += hmid @ w2_r[...]


def _s4b(h1, p):
    JD = DFF // 8
    return _call(
        _s4b_body,
        grid=(NRT, 8),
        in_specs=[pl.BlockSpec((RT, D), lambda r, j: (r, 0)),
                  pl.BlockSpec((1, D), lambda r, j: (0, 0)),
                  pl.BlockSpec((1, D), lambda r, j: (0, 0)),
                  pl.BlockSpec((D, JD), lambda r, j: (0, j)),
                  pl.BlockSpec((1, JD), lambda r, j: (0, j)),
                  pl.BlockSpec((JD, D), lambda r, j: (j, 0)),
                  pl.BlockSpec((1, D), lambda r, j: (0, 0))],
        out_specs=[pl.BlockSpec((RT, D), lambda r, j: (r, 0))],
        out_shape=[jax.ShapeDtypeStruct((N, D), jnp.float32)],
        scratch_shapes=[pltpu.VMEM((RT, D), jnp.float32)],
    )(h1, p['ln2_g'].reshape(1, D), p['ln2_b'].reshape(1, D),
      p['W1'], p['b1'].reshape(1, DFF), p['W2'], p['b2'].reshape(1, D))[0]


def _gather(tok, ktab, vtab):
    flat = tok.reshape(-1)
    return jnp.take(ktab, flat, axis=0), jnp.take(vtab, flat, axis=0)


def kernel(src, params):
    p = params
    src2 = src[0]
    q, k, v, glog = _s1(src2, p)

    q3 = q.reshape(N, H, DH).transpose(1, 0, 2)
    k3 = k.reshape(N, H, DH).transpose(1, 0, 2)
    v3 = v.reshape(N, H, DH).transpose(1, 0, 2)
    kp3 = jnp.concatenate([k3[:, :1], k3[:, :-1]], axis=1)
    vp3 = jnp.concatenate([v3[:, :1], v3[:, :-1]], axis=1)
    g3 = glog.reshape(N, H, 3).transpose(1, 0, 2)
    wo3 = p['Wo'].reshape(H, DH, D)

    kev = k3[:, 0::2]
    kod = k3[:, 1::2]
    vev = v3[:, 0::2]
    vod = v3[:, 1::2]
    ck3, cv3 = _s2(kev, kod, vev, vod, p)

    cout3, wout3, tok = _s3(q3, k3, v3, kp3, vp3, ck3, cv3)

    ktab = k3.reshape(H * N, DH)
    vtab = v3.reshape(H * N, DH)
    kselflat, vselflat = _gather(tok, ktab, vtab)
    ksel = kselflat.reshape(H, N * NSEL * BS, DH)
    vsel = vselflat.reshape(H, N * NSEL * BS, DH)

    sout3 = _s35(q3, tok, ksel, vsel)

    h1 = _s4a(cout3, sout3, wout3, g3, wo3, src2, p['bo'])
    out = _s4b(h1, p)
    return out.reshape(1, N, D)


# speedup vs baseline: 9.7987x; 2.3902x over previous
import functools

import jax
import jax.numpy as jnp
from jax import lax
from jax.experimental import pallas as pl
from jax.experimental.pallas import tpu as pltpu
from jax.experimental.pallas import tpu_sc as plsc

D = 1024
H = 16
DH = 64
DFF = 4096
N = 2048
BC = 4
SC = 2
BS = 4
NSEL = 2
RT = 256
NRT = N // RT
NC2 = N // SC
SCALE = DH ** -0.5

_call = pl.pallas_call


def _s1_body(src_r, wq_r, wk_r, wv_r, wg_r, g1_r, b1_r, bg_r,
             q_r, k_r, v_r, g_r):
    x = src_r[...]
    m = jnp.mean(x, -1, keepdims=True)
    va = jnp.mean((x - m) ** 2, -1, keepdims=True)
    xln = (x - m) / jnp.sqrt(va + 1e-5) * g1_r[...] + b1_r[...]
    q_r[...] = xln @ wq_r[...]
    k_r[...] = xln @ wk_r[...]
    v_r[...] = xln @ wv_r[...]
    g_r[...] = xln @ wg_r[...] + bg_r[...]


def _s1(src, p):
    full = lambda a, b: pl.BlockSpec((a, b), lambda i: (0, 0))
    return _call(
        _s1_body,
        grid=(NRT,),
        in_specs=[
            pl.BlockSpec((RT, D), lambda i: (i, 0)),
            full(D, H * DH), full(D, H * DH), full(D, H * DH), full(D, H * 3),
            full(1, D), full(1, D), full(1, H * 3),
        ],
        out_specs=[
            pl.BlockSpec((RT, H * DH), lambda i: (i, 0)),
            pl.BlockSpec((RT, H * DH), lambda i: (i, 0)),
            pl.BlockSpec((RT, H * DH), lambda i: (i, 0)),
            pl.BlockSpec((RT, H * 3), lambda i: (i, 0)),
        ],
        out_shape=[
            jax.ShapeDtypeStruct((N, H * DH), jnp.float32),
            jax.ShapeDtypeStruct((N, H * DH), jnp.float32),
            jax.ShapeDtypeStruct((N, H * DH), jnp.float32),
            jax.ShapeDtypeStruct((N, H * 3), jnp.float32),
        ],
    )(src, p['Wq'], p['Wk'], p['Wv'], p['Wg'],
      p['ln1_g'].reshape(1, D), p['ln1_b'].reshape(1, D),
      p['bg'].reshape(1, H * 3))


def _s2_body(kev_r, kod_r, vev_r, vod_r, wkc_r, bkc_r, wvc_r, bvc_r,
             ck_r, cv_r):
    kev = kev_r[0]
    kod = kod_r[0]
    vev = vev_r[0]
    vod = vod_r[0]
    z = jnp.zeros((1, DH), jnp.float32)
    kev1 = jnp.concatenate([kev[1:], z], axis=0)
    kod1 = jnp.concatenate([kod[1:], z], axis=0)
    vev1 = jnp.concatenate([vev[1:], z], axis=0)
    vod1 = jnp.concatenate([vod[1:], z], axis=0)
    ckc = jnp.concatenate([kev, kod, kev1, kod1], axis=1)
    cvc = jnp.concatenate([vev, vod, vev1, vod1], axis=1)
    ck_r[0] = ckc @ wkc_r[...] + bkc_r[...]
    cv_r[0] = cvc @ wvc_r[...] + bvc_r[...]


def _s2(kev, kod, vev, vod, p):
    blk = pl.BlockSpec((1, NC2, DH), lambda h: (h, 0, 0))
    full = lambda a, b: pl.BlockSpec((a, b), lambda h: (0, 0))
    return _call(
        _s2_body,
        grid=(H,),
        in_specs=[blk, blk, blk, blk,
                  full(BC * DH, DH), full(1, DH), full(BC * DH, DH), full(1, DH)],
        out_specs=[blk, blk],
        out_shape=[jax.ShapeDtypeStruct((H, NC2, DH), jnp.float32),
                   jax.ShapeDtypeStruct((H, NC2, DH), jnp.float32)],
    )(kev, kod, vev, vod, p['Wkc'], p['bkc'].reshape(1, DH),
      p['Wvc'], p['bvc'].reshape(1, DH))


def _s3_body(q_r, k_r, v_r, kp_r, vp_r, ck_r, cv_r,
             cout_r, wout_r, tok_r):
    h = pl.program_id(0)
    rt = pl.program_id(1)
    q = q_r[0]
    ck = ck_r[0]
    cv = cv_r[0]
    irow = rt * RT + lax.broadcasted_iota(jnp.int32, (RT, NC2), 0)
    col = lax.broadcasted_iota(jnp.int32, (RT, NC2), 1)

    clog = lax.dot_general(q, ck, (((1,), (1,)), ((), ()))) * SCALE
    cmask = (2 * col + BC - 1) <= irow
    clogm = jnp.where(cmask, clog, -1e9)
    rowmax = jnp.max(clogm, -1, keepdims=True)
    e = jnp.where(cmask, jnp.exp(clogm - rowmax), 0.0)
    den = jnp.sum(e, -1, keepdims=True)
    cattn = e / jnp.where(den == 0.0, 1.0, den)
    cout_r[0] = lax.dot_general(cattn, cv, (((1,), (0,)), ((), ())))

    esh = jnp.concatenate([e[:, 1:], jnp.zeros((RT, 1), jnp.float32)], axis=1)
    pair = e + esh
    scores = jnp.where((col % 2 == 0) & (2 * col <= irow), pair, -1.0)
    m1 = jnp.max(scores, -1, keepdims=True)
    i1 = jnp.min(jnp.where(scores == m1, col, NC2 * 4), -1, keepdims=True)
    scores2 = jnp.where(col == i1, -1.0, scores)
    m2 = jnp.max(scores2, -1, keepdims=True)
    i2 = jnp.min(jnp.where(scores2 == m2, col, NC2 * 4), -1, keepdims=True)
    sel1 = i1 // 2
    sel2 = i2 // 2
    c8 = lax.broadcasted_iota(jnp.int32, (RT, NSEL * BS), 1)
    blkid = jnp.where(c8 < BS, sel1, sel2)
    tok_r[0] = blkid * BS + (c8 % BS) + h * N

    k = k_r[0]
    v = v_r[0]
    kp = kp_r[0]
    vp = vp_r[0]
    ipc = rt * RT + lax.broadcasted_iota(jnp.int32, (RT, 1), 0)
    d1 = jnp.sum(q * k, -1, keepdims=True) * SCALE
    d0 = jnp.sum(q * kp, -1, keepdims=True) * SCALE
    valid0 = ipc >= 1
    d0m = jnp.where(valid0, d0, -1e9)
    mw = jnp.maximum(d0m, d1)
    e0 = jnp.where(valid0, jnp.exp(d0m - mw), 0.0)
    e1 = jnp.exp(d1 - mw)
    wout_r[0] = (e0 * vp + e1 * v) / (e0 + e1)


def _s3(q3, k3, v3, kp3, vp3, ck3, cv3):
    rblk = pl.BlockSpec((1, RT, DH), lambda h, r: (h, r, 0))
    cblk = pl.BlockSpec((1, NC2, DH), lambda h, r: (h, 0, 0))
    return _call(
        _s3_body,
        grid=(H, NRT),
        in_specs=[rblk, rblk, rblk, rblk, rblk, cblk, cblk],
        out_specs=[rblk, rblk,
                   pl.BlockSpec((1, RT, NSEL * BS), lambda h, r: (h, r, 0))],
        out_shape=[jax.ShapeDtypeStruct((H, N, DH), jnp.float32),
                   jax.ShapeDtypeStruct((H, N, DH), jnp.float32),
                   jax.ShapeDtypeStruct((H, N, NSEL * BS), jnp.int32)],
    )(q3, k3, v3, kp3, vp3, ck3, cv3)


def _s35_body(q_r, tok_r, kv_r, sout_r):
    h = pl.program_id(0)
    rt = pl.program_id(1)
    q = q_r[0]
    qz = jnp.concatenate([q, jnp.zeros((RT, DH), jnp.float32)], axis=1)
    kv = kv_r[0].reshape(RT, NSEL * BS, 2 * DH)
    tok = tok_r[0] - h * N
    ipc = rt * RT + lax.broadcasted_iota(jnp.int32, (RT, 1), 0)
    logs = []
    for j in range(NSEL * BS):
        lj = jnp.sum(qz * kv[:, j, :], -1, keepdims=True) * SCALE
        tm = tok[:, j:j + 1] <= ipc
        logs.append(jnp.where(tm, lj, -1e9))
    m = functools.reduce(jnp.maximum, logs)
    es = [jnp.exp(l - m) for l in logs]
    den = functools.reduce(jnp.add, es)
    acc = es[0] * kv[:, 0, :]
    for j in range(1, NSEL * BS):
        acc = acc + es[j] * kv[:, j, :]
    sout_r[0] = acc[:, DH:] / den


def _s35(q3, tok, kvsel):
    rblk = pl.BlockSpec((1, RT, DH), lambda h, r: (h, r, 0))
    return _call(
        _s35_body,
        grid=(H, NRT),
        in_specs=[rblk,
                  pl.BlockSpec((1, RT, NSEL * BS), lambda h, r: (h, r, 0)),
                  pl.BlockSpec((1, RT * NSEL * BS, 2 * DH),
                               lambda h, r: (h, r, 0))],
        out_specs=[rblk],
        out_shape=[jax.ShapeDtypeStruct((H, N, DH), jnp.float32)],
    )(q3, tok, kvsel)[0]


def _s4a_body(cout_r, sout_r, wout_r, g_r, wo_r, src_r, bo_r, h1_r):
    h = pl.program_id(1)
    g = jax.nn.sigmoid(g_r[0])
    o_h = (g[:, 0:1] * cout_r[0] + g[:, 1:2] * sout_r[0]
           + g[:, 2:3] * wout_r[0])
    part = o_h @ wo_r[0]

    @pl.when(h == 0)
    def _():
        h1_r[...] = src_r[...] + bo_r[...] + part

    @pl.when(h != 0)
    def _():
        h1_r[...] += part


def _s4a(cout3, sout3, wout3, g3, wo3, src, bo):
    rblk = pl.BlockSpec((1, RT, DH), lambda r, h: (h, r, 0))
    return _call(
        _s4a_body,
        grid=(NRT, H),
        in_specs=[rblk, rblk, rblk,
                  pl.BlockSpec((1, RT, 3), lambda r, h: (h, r, 0)),
                  pl.BlockSpec((1, DH, D), lambda r, h: (h, 0, 0)),
                  pl.BlockSpec((RT, D), lambda r, h: (r, 0)),
                  pl.BlockSpec((1, D), lambda r, h: (0, 0))],
        out_specs=[pl.BlockSpec((RT, D), lambda r, h: (r, 0))],
        out_shape=[jax.ShapeDtypeStruct((N, D), jnp.float32)],
    )(cout3, sout3, wout3, g3, wo3, src, bo.reshape(1, D))[0]


def _s4b_body(h1_r, g2_r, b2ln_r, w1_r, b1_r, w2_r, b2_r, out_r, y_scr):
    j = pl.program_id(1)

    @pl.when(j == 0)
    def _():
        x = h1_r[...]
        m = jnp.mean(x, -1, keepdims=True)
        va = jnp.mean((x - m) ** 2, -1, keepdims=True)
        y_scr[...] = (x - m) / jnp.sqrt(va + 1e-5) * g2_r[...] + b2ln_r[...]
        out_r[...] = x + b2_r[...]

    hmid = jax.nn.gelu(y_scr[...] @ w1_r[...] + b1_r[...])
    out_r[...] += hmid @ w2_r[...]


def _s4b(h1, p):
    JD = DFF // 8
    return _call(
        _s4b_body,
        grid=(NRT, 8),
        in_specs=[pl.BlockSpec((RT, D), lambda r, j: (r, 0)),
                  pl.BlockSpec((1, D), lambda r, j: (0, 0)),
                  pl.BlockSpec((1, D), lambda r, j: (0, 0)),
                  pl.BlockSpec((D, JD), lambda r, j: (0, j)),
                  pl.BlockSpec((1, JD), lambda r, j: (0, j)),
                  pl.BlockSpec((JD, D), lambda r, j: (j, 0)),
                  pl.BlockSpec((1, D), lambda r, j: (0, 0))],
        out_specs=[pl.BlockSpec((RT, D), lambda r, j: (r, 0))],
        out_shape=[jax.ShapeDtypeStruct((N, D), jnp.float32)],
        scratch_shapes=[pltpu.VMEM((RT, D), jnp.float32)],
    )(h1, p['ln2_g'].reshape(1, D), p['ln2_b'].reshape(1, D),
      p['W1'], p['b1'].reshape(1, DFF), p['W2'], p['b2'].reshape(1, D))[0]


_NROWS = H * N * NSEL * BS
_NW = 32
_CH = 128
_RPW = _NROWS // _NW
_NCHUNK = _RPW // _CH


def _scg_body(tok_hbm, kvtab_hbm, kvo_hbm, idx_v, rows_v, sem):
    wid = lax.axis_index("s") * 2 + lax.axis_index("c")
    base = wid * _RPW

    def body(c, carry):
        off = base + c * _CH
        pltpu.sync_copy(tok_hbm.at[pl.ds(off, _CH)], idx_v)
        pltpu.async_copy(kvtab_hbm.at[idx_v], rows_v, sem).wait()
        pltpu.sync_copy(rows_v, kvo_hbm.at[pl.ds(off, _CH)])
        return carry

    lax.fori_loop(0, _NCHUNK, body, 0)


def _gather(tok, kvtab):
    flat = tok.reshape(_NROWS)
    mesh = plsc.VectorSubcoreMesh(core_axis_name="c", subcore_axis_name="s")
    f = pl.kernel(
        _scg_body,
        mesh=mesh,
        out_type=jax.ShapeDtypeStruct((_NROWS, 2 * DH), jnp.float32),
        scratch_types=[pltpu.VMEM((_CH,), jnp.int32),
                       pltpu.VMEM((_CH, 2 * DH), jnp.float32),
                       pltpu.SemaphoreType.DMA],
    )
    return f(flat, kvtab)


def kernel(src, params):
    p = params
    src2 = src[0]
    q, k, v, glog = _s1(src2, p)

    q3 = q.reshape(N, H, DH).transpose(1, 0, 2)
    k3 = k.reshape(N, H, DH).transpose(1, 0, 2)
    v3 = v.reshape(N, H, DH).transpose(1, 0, 2)
    kp3 = jnp.concatenate([k3[:, :1], k3[:, :-1]], axis=1)
    vp3 = jnp.concatenate([v3[:, :1], v3[:, :-1]], axis=1)
    g3 = glog.reshape(N, H, 3).transpose(1, 0, 2)
    wo3 = p['Wo'].reshape(H, DH, D)

    kev = k3[:, 0::2]
    kod = k3[:, 1::2]
    vev = v3[:, 0::2]
    vod = v3[:, 1::2]
    ck3, cv3 = _s2(kev, kod, vev, vod, p)

    cout3, wout3, tok = _s3(q3, k3, v3, kp3, vp3, ck3, cv3)

    kvtab = jnp.concatenate([k3, v3], axis=2).reshape(H * N, 2 * DH)
    kvsel = _gather(tok, kvtab).reshape(H, N * NSEL * BS, 2 * DH)

    sout3 = _s35(q3, tok, kvsel)

    h1 = _s4a(cout3, sout3, wout3, g3, wo3, src2, p['bo'])
    out = _s4b(h1, p)
    return out.reshape(1, N, D)


# speedup vs baseline: 13.9536x; 1.4240x over previous
import functools

import jax
import jax.numpy as jnp
from jax import lax
from jax.experimental import pallas as pl
from jax.experimental.pallas import tpu as pltpu
from jax.experimental.pallas import tpu_sc as plsc

D = 1024
H = 16
DH = 64
DFF = 4096
N = 2048
BC = 4
SC = 2
BS = 4
NSEL = 2
RT = 256
NRT = N // RT
NC2 = N // SC
NB = N // BS
SCALE = DH ** -0.5

_call = pl.pallas_call


def _s1_body(src_r, wq_r, wkv_r, wg_r, g1_r, b1_r, bg_r,
             q_r, kv_r, g_r):
    x = src_r[...]
    m = jnp.mean(x, -1, keepdims=True)
    va = jnp.mean((x - m) ** 2, -1, keepdims=True)
    xln = (x - m) / jnp.sqrt(va + 1e-5) * g1_r[...] + b1_r[...]
    q_r[...] = xln @ wq_r[...]
    kv_r[...] = xln @ wkv_r[...]
    g_r[...] = xln @ wg_r[...] + bg_r[...]


def _s1(src, wq, wkv, wg_r, ln1g, ln1b, bg_r):
    full = lambda a, b: pl.BlockSpec((a, b), lambda i: (0, 0))
    return _call(
        _s1_body,
        grid=(NRT,),
        in_specs=[
            pl.BlockSpec((RT, D), lambda i: (i, 0)),
            full(D, H * DH), full(D, 2 * H * DH), full(D, H * 3),
            full(1, D), full(1, D), full(1, H * 3),
        ],
        out_specs=[
            pl.BlockSpec((RT, H * DH), lambda i: (i, 0)),
            pl.BlockSpec((RT, 2 * H * DH), lambda i: (i, 0)),
            pl.BlockSpec((RT, H * 3), lambda i: (i, 0)),
        ],
        out_shape=[
            jax.ShapeDtypeStruct((N, H * DH), jnp.float32),
            jax.ShapeDtypeStruct((N, 2 * H * DH), jnp.float32),
            jax.ShapeDtypeStruct((N, H * 3), jnp.float32),
        ],
    )(src, wq, wkv, wg_r, ln1g.reshape(1, D), ln1b.reshape(1, D),
      bg_r.reshape(1, H * 3))


def _s2_body(ev_r, od_r, wkc_r, bkc_r, wvc_r, bvc_r, ck_r, cv_r):
    ev = ev_r[...]
    od = od_r[...]
    kev, vev = ev[:, :DH], ev[:, DH:]
    kod, vod = od[:, :DH], od[:, DH:]
    z = jnp.zeros((1, DH), jnp.float32)
    kev1 = jnp.concatenate([kev[1:], z], axis=0)
    kod1 = jnp.concatenate([kod[1:], z], axis=0)
    vev1 = jnp.concatenate([vev[1:], z], axis=0)
    vod1 = jnp.concatenate([vod[1:], z], axis=0)
    ckc = jnp.concatenate([kev, kod, kev1, kod1], axis=1)
    cvc = jnp.concatenate([vev, vod, vev1, vod1], axis=1)
    ck_r[0] = ckc @ wkc_r[...] + bkc_r[...]
    cv_r[0] = cvc @ wvc_r[...] + bvc_r[...]


def _s2(kv2, p):
    blk = pl.BlockSpec((1, NC2, DH), lambda h: (h, 0, 0))
    full = lambda a, b: pl.BlockSpec((a, b), lambda h: (0, 0))
    return _call(
        _s2_body,
        grid=(H,),
        in_specs=[pl.BlockSpec((NC2, 2 * DH), lambda h: (0, h)),
                  pl.BlockSpec((NC2, 2 * DH), lambda h: (0, H + h)),
                  full(BC * DH, DH), full(1, DH), full(BC * DH, DH), full(1, DH)],
        out_specs=[blk, blk],
        out_shape=[jax.ShapeDtypeStruct((H, NC2, DH), jnp.float32),
                   jax.ShapeDtypeStruct((H, NC2, DH), jnp.float32)],
    )(kv2, kv2, p['Wkc'], p['bkc'].reshape(1, DH),
      p['Wvc'], p['bvc'].reshape(1, DH))


def _s3_body(q_r, kv_r, kvp_r, ck_r, cv_r, cout_r, wout_r, gidx_r):
    h2 = pl.program_id(0)
    rt = pl.program_id(1)
    irow = rt * RT + lax.broadcasted_iota(jnp.int32, (RT, NC2), 0)
    col = lax.broadcasted_iota(jnp.int32, (RT, NC2), 1)
    cmask = (2 * col + BC - 1) <= irow
    scond = (col % 2 == 0) & (2 * col <= irow)
    c2 = lax.broadcasted_iota(jnp.int32, (RT, NSEL), 1)
    ipc = rt * RT + lax.broadcasted_iota(jnp.int32, (RT, 1), 0)
    valid0 = ipc >= 1

    for p in range(2):
        q = q_r[:, p * DH:(p + 1) * DH]
        ck = ck_r[p]
        cv = cv_r[p]
        clog = lax.dot_general(q, ck, (((1,), (1,)), ((), ()))) * SCALE
        clogm = jnp.where(cmask, clog, -1e9)
        rowmax = jnp.max(clogm, -1, keepdims=True)
        e = jnp.where(cmask, jnp.exp(clogm - rowmax), 0.0)
        den = jnp.sum(e, -1, keepdims=True)
        cattn = e / jnp.where(den == 0.0, 1.0, den)
        cout_r[p] = lax.dot_general(cattn, cv, (((1,), (0,)), ((), ())))

        esh = jnp.concatenate([e[:, 1:], jnp.zeros((RT, 1), jnp.float32)],
                              axis=1)
        pair = e + esh
        scores = jnp.where(scond, pair, -1.0)
        m1 = jnp.max(scores, -1, keepdims=True)
        i1 = jnp.min(jnp.where(scores == m1, col, NC2 * 4), -1, keepdims=True)
        scores2 = jnp.where(col == i1, -1.0, scores)
        m2 = jnp.max(scores2, -1, keepdims=True)
        i2 = jnp.min(jnp.where(scores2 == m2, col, NC2 * 4), -1, keepdims=True)
        sel1 = i1 // 2
        sel2 = i2 // 2
        gidx_r[p] = jnp.where(c2 == 0, sel1, sel2) + (h2 * 2 + p) * NB

        k = kv_r[:, p * 2 * DH:p * 2 * DH + DH]
        v = kv_r[:, p * 2 * DH + DH:(p + 1) * 2 * DH]
        kp = kvp_r[:, p * 2 * DH:p * 2 * DH + DH]
        vp = kvp_r[:, p * 2 * DH + DH:(p + 1) * 2 * DH]
        d1 = jnp.sum(q * k, -1, keepdims=True) * SCALE
        d0 = jnp.sum(q * kp, -1, keepdims=True) * SCALE
        d0m = jnp.where(valid0, d0, -1e9)
        mw = jnp.maximum(d0m, d1)
        e0 = jnp.where(valid0, jnp.exp(d0m - mw), 0.0)
        e1 = jnp.exp(d1 - mw)
        wout_r[p] = (e0 * vp + e1 * v) / (e0 + e1)


def _s3(q, kv, kvp, ck3, cv3):
    hblk = pl.BlockSpec((2, RT, DH), lambda h2, r: (h2, r, 0))
    return _call(
        _s3_body,
        grid=(H // 2, NRT),
        in_specs=[pl.BlockSpec((RT, 2 * DH), lambda h2, r: (r, h2)),
                  pl.BlockSpec((RT, 4 * DH), lambda h2, r: (r, h2)),
                  pl.BlockSpec((RT, 4 * DH), lambda h2, r: (r, h2)),
                  pl.BlockSpec((2, NC2, DH), lambda h2, r: (h2, 0, 0)),
                  pl.BlockSpec((2, NC2, DH), lambda h2, r: (h2, 0, 0))],
        out_specs=[hblk, hblk,
                   pl.BlockSpec((2, RT, NSEL), lambda h2, r: (h2, r, 0))],
        out_shape=[jax.ShapeDtypeStruct((H, N, DH), jnp.float32),
                   jax.ShapeDtypeStruct((H, N, DH), jnp.float32),
                   jax.ShapeDtypeStruct((H, N, NSEL), jnp.int32)],
    )(q, kv, kvp, ck3, cv3)


_NROWS = H * N * NSEL
_RW = BS * 2 * DH
_NW = 32
_CH = 64
_RPW = _NROWS // _NW
_NCHUNK = _RPW // _CH


def _scg_body(gidx_hbm, tab_hbm, out_hbm, idx_v, rows_v, sem):
    wid = lax.axis_index("s") * 2 + lax.axis_index("c")
    base = wid * _RPW

    def body(c, carry):
        off = base + c * _CH
        pltpu.sync_copy(gidx_hbm.at[pl.ds(off, _CH)], idx_v)
        pltpu.async_copy(tab_hbm.at[idx_v], rows_v, sem).wait()
        pltpu.sync_copy(rows_v, out_hbm.at[pl.ds(off, _CH)])
        return carry

    lax.fori_loop(0, _NCHUNK, body, 0)


def _gather(gidxt, tab):
    mesh = plsc.VectorSubcoreMesh(core_axis_name="c", subcore_axis_name="s")
    f = pl.kernel(
        _scg_body,
        mesh=mesh,
        out_type=jax.ShapeDtypeStruct((_NROWS, _RW), jnp.float32),
        scratch_types=[pltpu.VMEM((_CH,), jnp.int32),
                       pltpu.VMEM((_CH, _RW), jnp.float32),
                       pltpu.SemaphoreType.DMA],
    )
    return f(gidxt, tab)


def _s35_body(q_r, gidx_r, kv_r, sout_r):
    h2 = pl.program_id(0)
    rt = pl.program_id(1)
    ipc = rt * RT + lax.broadcasted_iota(jnp.int32, (RT, 1), 0)
    zpad = jnp.zeros((RT, DH), jnp.float32)
    for p in range(2):
        q = q_r[:, p * DH:(p + 1) * DH]
        qz = jnp.concatenate([q, zpad], axis=1)
        sel = gidx_r[p] - (h2 * 2 + p) * NB
        logs = []
        toks = []
        for s in range(NSEL):
            sel_s = sel[:, s:s + 1]
            for t in range(BS):
                kv_t = kv_r[p, s, :, t * 2 * DH:(t + 1) * 2 * DH]
                lj = jnp.sum(qz * kv_t, -1, keepdims=True) * SCALE
                tm = (sel_s * BS + t) <= ipc
                logs.append(jnp.where(tm, lj, -1e9))
                toks.append(kv_t)
        m = functools.reduce(jnp.maximum, logs)
        es = [jnp.exp(l - m) for l in logs]
        den = functools.reduce(jnp.add, es)
        acc = es[0] * toks[0]
        for j in range(1, NSEL * BS):
            acc = acc + es[j] * toks[j]
        sout_r[p] = acc[:, DH:] / den


def _s35(q, gidx, kvsel):
    return _call(
        _s35_body,
        grid=(H // 2, NRT),
        in_specs=[pl.BlockSpec((RT, 2 * DH), lambda h2, r: (r, h2)),
                  pl.BlockSpec((2, RT, NSEL), lambda h2, r: (h2, r, 0)),
                  pl.BlockSpec((2, NSEL, RT, _RW), lambda h2, r: (h2, 0, r, 0))],
        out_specs=[pl.BlockSpec((2, RT, DH), lambda h2, r: (h2, r, 0))],
        out_shape=[jax.ShapeDtypeStruct((H, N, DH), jnp.float32)],
    )(q, gidx, kvsel)[0]


def _s4a_body(cout_r, sout_r, wout_r, g_r, ex_r, wo_r, src_r, bo_r, h1_r):
    j = pl.program_id(1)
    gx = jax.nn.sigmoid(g_r[...]) @ ex_r[0]
    cat = lambda x_r: jnp.concatenate([x_r[0], x_r[1], x_r[2], x_r[3]], axis=1)
    o4 = (gx[:, 0:4 * DH] * cat(cout_r)
          + gx[:, 4 * DH:8 * DH] * cat(sout_r)
          + gx[:, 8 * DH:12 * DH] * cat(wout_r))
    part = o4 @ wo_r[...]

    @pl.when(j == 0)
    def _():
        h1_r[...] = src_r[...] + bo_r[...] + part

    @pl.when(j != 0)
    def _():
        h1_r[...] += part


def _s4a(cout, sout, wout, g, ex, wo, src, bo):
    hblk = pl.BlockSpec((4, RT, DH), lambda r, j: (j, r, 0))
    return _call(
        _s4a_body,
        grid=(NRT, 4),
        in_specs=[hblk, hblk, hblk,
                  pl.BlockSpec((RT, H * 3), lambda r, j: (r, 0)),
                  pl.BlockSpec((1, H * 3, 12 * DH), lambda r, j: (j, 0, 0)),
                  pl.BlockSpec((4 * DH, D), lambda r, j: (j, 0)),
                  pl.BlockSpec((RT, D), lambda r, j: (r, 0)),
                  pl.BlockSpec((1, D), lambda r, j: (0, 0))],
        out_specs=[pl.BlockSpec((RT, D), lambda r, j: (r, 0))],
        out_shape=[jax.ShapeDtypeStruct((N, D), jnp.float32)],
    )(cout, sout, wout, g, ex, wo, src, bo.reshape(1, D))[0]


def _s4b_body(h1_r, g2_r, b2ln_r, w1_r, b1_r, w2_r, b2_r, out_r, y_scr):
    j = pl.program_id(1)

    @pl.when(j == 0)
    def _():
        x = h1_r[...]
        m = jnp.mean(x, -1, keepdims=True)
        va = jnp.mean((x - m) ** 2, -1, keepdims=True)
        y_scr[...] = (x - m) / jnp.sqrt(va + 1e-5) * g2_r[...] + b2ln_r[...]
        out_r[...] = x + b2_r[...]

    hmid = jax.nn.gelu(y_scr[...] @ w1_r[...] + b1_r[...])
    out_r[...] += hmid @ w2_r[...]


def _s4b(h1, p):
    JD = DFF // 8
    return _call(
        _s4b_body,
        grid=(NRT, 8),
        in_specs=[pl.BlockSpec((RT, D), lambda r, j: (r, 0)),
                  pl.BlockSpec((1, D), lambda r, j: (0, 0)),
                  pl.BlockSpec((1, D), lambda r, j: (0, 0)),
                  pl.BlockSpec((D, JD), lambda r, j: (0, j)),
                  pl.BlockSpec((1, JD), lambda r, j: (0, j)),
                  pl.BlockSpec((JD, D), lambda r, j: (j, 0)),
                  pl.BlockSpec((1, D), lambda r, j: (0, 0))],
        out_specs=[pl.BlockSpec((RT, D), lambda r, j: (r, 0))],
        out_shape=[jax.ShapeDtypeStruct((N, D), jnp.float32)],
        scratch_shapes=[pltpu.VMEM((RT, D), jnp.float32)],
    )(h1, p['ln2_g'].reshape(1, D), p['ln2_b'].reshape(1, D),
      p['W1'], p['b1'].reshape(1, DFF), p['W2'], p['b2'].reshape(1, D))[0]


def kernel(src, params):
    p = params
    src2 = src[0]

    wkv = jnp.concatenate([p['Wk'].reshape(D, H, DH),
                           p['Wv'].reshape(D, H, DH)], axis=2).reshape(D, 2 * H * DH)
    wg_r = p['Wg'].reshape(D, H, 3).transpose(0, 2, 1).reshape(D, H * 3)
    bg_r = p['bg'].reshape(H, 3).T.reshape(H * 3)
    gidx48 = jnp.arange(H * 3)
    cidx = jnp.arange(12 * DH)
    ex = (gidx48[None, :, None]
          == ((cidx[None, None, :] // (4 * DH)) * H
              + 4 * jnp.arange(4)[:, None, None]
              + (cidx[None, None, :] % (4 * DH)) // DH)).astype(jnp.float32)

    q, kv, glog = _s1(src2, p['Wq'], wkv, wg_r, p['ln1_g'], p['ln1_b'], bg_r)

    kvp = jnp.concatenate([kv[:1], kv[:-1]], axis=0)
    kv2 = kv.reshape(NC2, 2 * 2 * H * DH)

    ck3, cv3 = _s2(kv2, p)
    cout, wout, gidx = _s3(q, kv, kvp, ck3, cv3)

    tab = kv.reshape(NB, BS, H, 2 * DH).transpose(2, 0, 1, 3).reshape(H * NB, _RW)
    gidxt = gidx.transpose(0, 2, 1).reshape(_NROWS)
    kvsel = _gather(gidxt, tab).reshape(H, NSEL, N, _RW)

    sout = _s35(q, gidx, kvsel)

    h1 = _s4a(cout, sout, wout, glog, ex, p['Wo'], src2, p['bo'])
    out = _s4b(h1, p)
    return out.reshape(1, N, D)


# speedup vs baseline: 14.6289x; 1.0484x over previous
import functools

import jax
import jax.numpy as jnp
from jax import lax
from jax.experimental import pallas as pl
from jax.experimental.pallas import tpu as pltpu
from jax.experimental.pallas import tpu_sc as plsc

D = 1024
H = 16
DH = 64
DFF = 4096
N = 2048
BC = 4
SC = 2
BS = 4
NSEL = 2
RT = 256
NRT = N // RT
RT3 = 512
RT35 = 512
NC2 = N // SC
NB = N // BS
SCALE = DH ** -0.5

_call = pl.pallas_call


def _s1_body(src_r, wq_r, wkv_r, wg_r, g1_r, b1_r, bg_r,
             q_r, kv_r, g_r):
    x = src_r[...]
    m = jnp.mean(x, -1, keepdims=True)
    va = jnp.mean((x - m) ** 2, -1, keepdims=True)
    xln = (x - m) / jnp.sqrt(va + 1e-5) * g1_r[...] + b1_r[...]
    q_r[...] = xln @ wq_r[...]
    kv_r[...] = xln @ wkv_r[...]
    g_r[...] = xln @ wg_r[...] + bg_r[...]


def _s1(src, wq, wkv, wg_r, ln1g, ln1b, bg_r):
    full = lambda a, b: pl.BlockSpec((a, b), lambda i: (0, 0))
    return _call(
        _s1_body,
        grid=(NRT,),
        in_specs=[
            pl.BlockSpec((RT, D), lambda i: (i, 0)),
            full(D, H * DH), full(D, 2 * H * DH), full(D, H * 3),
            full(1, D), full(1, D), full(1, H * 3),
        ],
        out_specs=[
            pl.BlockSpec((RT, H * DH), lambda i: (i, 0)),
            pl.BlockSpec((RT, 2 * H * DH), lambda i: (i, 0)),
            pl.BlockSpec((RT, H * 3), lambda i: (i, 0)),
        ],
        out_shape=[
            jax.ShapeDtypeStruct((N, H * DH), jnp.float32),
            jax.ShapeDtypeStruct((N, 2 * H * DH), jnp.float32),
            jax.ShapeDtypeStruct((N, H * 3), jnp.float32),
        ],
    )(src, wq, wkv, wg_r, ln1g.reshape(1, D), ln1b.reshape(1, D),
      bg_r.reshape(1, H * 3))


def _s2_body(ev_r, od_r, wkc_r, bkc_r, wvc_r, bvc_r, ck_r, cv_r):
    ev = ev_r[...]
    od = od_r[...]
    kev, vev = ev[:, :DH], ev[:, DH:]
    kod, vod = od[:, :DH], od[:, DH:]
    z = jnp.zeros((1, DH), jnp.float32)
    kev1 = jnp.concatenate([kev[1:], z], axis=0)
    kod1 = jnp.concatenate([kod[1:], z], axis=0)
    vev1 = jnp.concatenate([vev[1:], z], axis=0)
    vod1 = jnp.concatenate([vod[1:], z], axis=0)
    ckc = jnp.concatenate([kev, kod, kev1, kod1], axis=1)
    cvc = jnp.concatenate([vev, vod, vev1, vod1], axis=1)
    ck_r[0] = ckc @ wkc_r[...] + bkc_r[...]
    cv_r[0] = cvc @ wvc_r[...] + bvc_r[...]


def _s2(kv2, p):
    blk = pl.BlockSpec((1, NC2, DH), lambda h: (h, 0, 0))
    full = lambda a, b: pl.BlockSpec((a, b), lambda h: (0, 0))
    return _call(
        _s2_body,
        grid=(H,),
        in_specs=[pl.BlockSpec((NC2, 2 * DH), lambda h: (0, h)),
                  pl.BlockSpec((NC2, 2 * DH), lambda h: (0, H + h)),
                  full(BC * DH, DH), full(1, DH), full(BC * DH, DH), full(1, DH)],
        out_specs=[blk, blk],
        out_shape=[jax.ShapeDtypeStruct((H, NC2, DH), jnp.float32),
                   jax.ShapeDtypeStruct((H, NC2, DH), jnp.float32)],
    )(kv2, kv2, p['Wkc'], p['bkc'].reshape(1, DH),
      p['Wvc'], p['bvc'].reshape(1, DH))


def _s3_body(q_r, kv_r, kvp_r, ck_r, cv_r, cout_r, wout_r, gidx_r):
    h2 = pl.program_id(0)
    rt = pl.program_id(1)
    irow = rt * RT3 + lax.broadcasted_iota(jnp.int32, (RT3, NC2), 0)
    col = lax.broadcasted_iota(jnp.int32, (RT3, NC2), 1)
    cmask = (2 * col + BC - 1) <= irow
    scond = (col % 2 == 0) & (2 * col <= irow)
    c2 = lax.broadcasted_iota(jnp.int32, (RT3, NSEL), 1)
    ipc = rt * RT3 + lax.broadcasted_iota(jnp.int32, (RT3, 1), 0)
    valid0 = ipc >= 1
    anyvis = ipc >= (BC - 1)

    for p in range(2):
        q = q_r[:, p * DH:(p + 1) * DH]
        ck = ck_r[p]
        cv = cv_r[p]
        clog = lax.dot_general(q, ck, (((1,), (1,)), ((), ()))) * SCALE
        clogm = jnp.where(cmask, clog, -1e9)
        rowmax = jnp.max(clogm, -1, keepdims=True)
        e = jnp.exp(clogm - rowmax)
        den = jnp.sum(e, -1, keepdims=True)
        cout = lax.dot_general(e, cv, (((1,), (0,)), ((), ())))
        cout_r[p] = jnp.where(anyvis, cout / den, 0.0)

        esh = jnp.concatenate([e[:, 1:], jnp.zeros((RT3, 1), jnp.float32)],
                              axis=1)
        pair = e + esh
        scores = jnp.where(scond, pair, -1.0)
        m1 = jnp.max(scores, -1, keepdims=True)
        i1 = jnp.min(jnp.where(scores == m1, col, NC2 * 4), -1, keepdims=True)
        scores2 = jnp.where(col == i1, -1.0, scores)
        m2 = jnp.max(scores2, -1, keepdims=True)
        i2 = jnp.min(jnp.where(scores2 == m2, col, NC2 * 4), -1, keepdims=True)
        sel1 = i1 // 2
        sel2 = i2 // 2
        gidx_r[p] = jnp.where(c2 == 0, sel1, sel2) + (h2 * 2 + p) * NB

        k = kv_r[:, p * 2 * DH:p * 2 * DH + DH]
        v = kv_r[:, p * 2 * DH + DH:(p + 1) * 2 * DH]
        kp = kvp_r[:, p * 2 * DH:p * 2 * DH + DH]
        vp = kvp_r[:, p * 2 * DH + DH:(p + 1) * 2 * DH]
        d1 = jnp.sum(q * k, -1, keepdims=True) * SCALE
        d0 = jnp.sum(q * kp, -1, keepdims=True) * SCALE
        d0m = jnp.where(valid0, d0, -1e9)
        mw = jnp.maximum(d0m, d1)
        e0 = jnp.where(valid0, jnp.exp(d0m - mw), 0.0)
        e1 = jnp.exp(d1 - mw)
        wout_r[p] = (e0 * vp + e1 * v) / (e0 + e1)


def _s3(q, kv, kvp, ck3, cv3):
    hblk = pl.BlockSpec((2, RT3, DH), lambda h2, r: (h2, r, 0))
    return _call(
        _s3_body,
        grid=(H // 2, N // RT3),
        in_specs=[pl.BlockSpec((RT3, 2 * DH), lambda h2, r: (r, h2)),
                  pl.BlockSpec((RT3, 4 * DH), lambda h2, r: (r, h2)),
                  pl.BlockSpec((RT3, 4 * DH), lambda h2, r: (r, h2)),
                  pl.BlockSpec((2, NC2, DH), lambda h2, r: (h2, 0, 0)),
                  pl.BlockSpec((2, NC2, DH), lambda h2, r: (h2, 0, 0))],
        out_specs=[hblk, hblk,
                   pl.BlockSpec((2, RT3, NSEL), lambda h2, r: (h2, r, 0))],
        out_shape=[jax.ShapeDtypeStruct((H, N, DH), jnp.float32),
                   jax.ShapeDtypeStruct((H, N, DH), jnp.float32),
                   jax.ShapeDtypeStruct((H, N, NSEL), jnp.int32)],
    )(q, kv, kvp, ck3, cv3)


_NROWS = H * N * NSEL
_RW = BS * 2 * DH
_NW = 32
_CH = 64
_RPW = _NROWS // _NW
_NCHUNK = _RPW // _CH


def _scg_body(gidx_hbm, tab_hbm, out_hbm,
              idx0_v, idx1_v, rows0_v, rows1_v, sem0, sem1):
    wid = lax.axis_index("s") * 2 + lax.axis_index("c")
    base = wid * _RPW

    def body(c2, carry):
        off0 = base + (2 * c2) * _CH
        off1 = off0 + _CH
        pltpu.sync_copy(gidx_hbm.at[pl.ds(off0, _CH)], idx0_v)
        g0 = pltpu.async_copy(tab_hbm.at[idx0_v], rows0_v, sem0)
        pltpu.sync_copy(gidx_hbm.at[pl.ds(off1, _CH)], idx1_v)
        g1 = pltpu.async_copy(tab_hbm.at[idx1_v], rows1_v, sem1)
        g0.wait()
        pltpu.sync_copy(rows0_v, out_hbm.at[pl.ds(off0, _CH)])
        g1.wait()
        pltpu.sync_copy(rows1_v, out_hbm.at[pl.ds(off1, _CH)])
        return carry

    lax.fori_loop(0, _NCHUNK // 2, body, 0)


def _gather(gidxt, tab):
    mesh = plsc.VectorSubcoreMesh(core_axis_name="c", subcore_axis_name="s")
    f = pl.kernel(
        _scg_body,
        mesh=mesh,
        out_type=jax.ShapeDtypeStruct((_NROWS, _RW), jnp.float32),
        scratch_types=[pltpu.VMEM((_CH,), jnp.int32),
                       pltpu.VMEM((_CH,), jnp.int32),
                       pltpu.VMEM((_CH, _RW), jnp.float32),
                       pltpu.VMEM((_CH, _RW), jnp.float32),
                       pltpu.SemaphoreType.DMA,
                       pltpu.SemaphoreType.DMA],
    )
    return f(gidxt, tab)


def _s35_body(q_r, gidx_r, kv_r, sout_r):
    h2 = pl.program_id(0)
    rt = pl.program_id(1)
    ipc = rt * RT35 + lax.broadcasted_iota(jnp.int32, (RT35, 1), 0)
    zpad = jnp.zeros((RT35, DH), jnp.float32)
    for p in range(2):
        q = q_r[:, p * DH:(p + 1) * DH]
        qz = jnp.concatenate([q, zpad], axis=1)
        sel = gidx_r[p] - (h2 * 2 + p) * NB
        logs = []
        toks = []
        for s in range(NSEL):
            sel_s = sel[:, s:s + 1]
            for t in range(BS):
                kv_t = kv_r[p, s, :, t * 2 * DH:(t + 1) * 2 * DH]
                lj = jnp.sum(qz * kv_t, -1, keepdims=True) * SCALE
                tm = (sel_s * BS + t) <= ipc
                logs.append(jnp.where(tm, lj, -1e9))
                toks.append(kv_t)
        m = functools.reduce(jnp.maximum, logs)
        es = [jnp.exp(l - m) for l in logs]
        den = functools.reduce(jnp.add, es)
        acc = es[0] * toks[0]
        for j in range(1, NSEL * BS):
            acc = acc + es[j] * toks[j]
        sout_r[p] = acc[:, DH:] / den


def _s35(q, gidx, kvsel):
    return _call(
        _s35_body,
        grid=(H // 2, N // RT35),
        in_specs=[pl.BlockSpec((RT35, 2 * DH), lambda h2, r: (r, h2)),
                  pl.BlockSpec((2, RT35, NSEL), lambda h2, r: (h2, r, 0)),
                  pl.BlockSpec((2, NSEL, RT35, _RW),
                               lambda h2, r: (h2, 0, r, 0))],
        out_specs=[pl.BlockSpec((2, RT35, DH), lambda h2, r: (h2, r, 0))],
        out_shape=[jax.ShapeDtypeStruct((H, N, DH), jnp.float32)],
    )(q, gidx, kvsel)[0]


def _s4a_body(cout_r, sout_r, wout_r, g_r, ex_r, wo_r, src_r, bo_r, h1_r):
    j = pl.program_id(1)
    gx = jax.nn.sigmoid(g_r[...]) @ ex_r[0]
    cat = lambda x_r: jnp.concatenate([x_r[0], x_r[1], x_r[2], x_r[3]], axis=1)
    o4 = (gx[:, 0:4 * DH] * cat(cout_r)
          + gx[:, 4 * DH:8 * DH] * cat(sout_r)
          + gx[:, 8 * DH:12 * DH] * cat(wout_r))
    part = o4 @ wo_r[...]

    @pl.when(j == 0)
    def _():
        h1_r[...] = src_r[...] + bo_r[...] + part

    @pl.when(j != 0)
    def _():
        h1_r[...] += part


def _s4a(cout, sout, wout, g, ex, wo, src, bo):
    hblk = pl.BlockSpec((4, RT, DH), lambda r, j: (j, r, 0))
    return _call(
        _s4a_body,
        grid=(NRT, 4),
        in_specs=[hblk, hblk, hblk,
                  pl.BlockSpec((RT, H * 3), lambda r, j: (r, 0)),
                  pl.BlockSpec((1, H * 3, 12 * DH), lambda r, j: (j, 0, 0)),
                  pl.BlockSpec((4 * DH, D), lambda r, j: (j, 0)),
                  pl.BlockSpec((RT, D), lambda r, j: (r, 0)),
                  pl.BlockSpec((1, D), lambda r, j: (0, 0))],
        out_specs=[pl.BlockSpec((RT, D), lambda r, j: (r, 0))],
        out_shape=[jax.ShapeDtypeStruct((N, D), jnp.float32)],
    )(cout, sout, wout, g, ex, wo, src, bo.reshape(1, D))[0]


def _s4b_body(h1_r, g2_r, b2ln_r, w1_r, b1_r, w2_r, b2_r, out_r, y_scr):
    j = pl.program_id(1)

    @pl.when(j == 0)
    def _():
        x = h1_r[...]
        m = jnp.mean(x, -1, keepdims=True)
        va = jnp.mean((x - m) ** 2, -1, keepdims=True)
        y_scr[...] = (x - m) / jnp.sqrt(va + 1e-5) * g2_r[...] + b2ln_r[...]
        out_r[...] = x + b2_r[...]

    hmid = jax.nn.gelu(y_scr[...] @ w1_r[...] + b1_r[...])
    out_r[...] += hmid @ w2_r[...]


def _s4b(h1, p):
    JD = DFF // 8
    return _call(
        _s4b_body,
        grid=(NRT, 8),
        in_specs=[pl.BlockSpec((RT, D), lambda r, j: (r, 0)),
                  pl.BlockSpec((1, D), lambda r, j: (0, 0)),
                  pl.BlockSpec((1, D), lambda r, j: (0, 0)),
                  pl.BlockSpec((D, JD), lambda r, j: (0, j)),
                  pl.BlockSpec((1, JD), lambda r, j: (0, j)),
                  pl.BlockSpec((JD, D), lambda r, j: (j, 0)),
                  pl.BlockSpec((1, D), lambda r, j: (0, 0))],
        out_specs=[pl.BlockSpec((RT, D), lambda r, j: (r, 0))],
        out_shape=[jax.ShapeDtypeStruct((N, D), jnp.float32)],
        scratch_shapes=[pltpu.VMEM((RT, D), jnp.float32)],
    )(h1, p['ln2_g'].reshape(1, D), p['ln2_b'].reshape(1, D),
      p['W1'], p['b1'].reshape(1, DFF), p['W2'], p['b2'].reshape(1, D))[0]


def kernel(src, params):
    p = params
    src2 = src[0]

    wkv = jnp.concatenate([p['Wk'].reshape(D, H, DH),
                           p['Wv'].reshape(D, H, DH)], axis=2).reshape(D, 2 * H * DH)
    wg_r = p['Wg'].reshape(D, H, 3).transpose(0, 2, 1).reshape(D, H * 3)
    bg_r = p['bg'].reshape(H, 3).T.reshape(H * 3)
    gidx48 = jnp.arange(H * 3)
    cidx = jnp.arange(12 * DH)
    ex = (gidx48[None, :, None]
          == ((cidx[None, None, :] // (4 * DH)) * H
              + 4 * jnp.arange(4)[:, None, None]
              + (cidx[None, None, :] % (4 * DH)) // DH)).astype(jnp.float32)

    q, kv, glog = _s1(src2, p['Wq'], wkv, wg_r, p['ln1_g'], p['ln1_b'], bg_r)

    kvp = jnp.concatenate([kv[:1], kv[:-1]], axis=0)
    kv2 = kv.reshape(NC2, 2 * 2 * H * DH)

    ck3, cv3 = _s2(kv2, p)
    cout, wout, gidx = _s3(q, kv, kvp, ck3, cv3)

    tab = kv.reshape(NB, BS, H, 2 * DH).transpose(2, 0, 1, 3).reshape(H * NB, _RW)
    gidxt = gidx.transpose(0, 2, 1).reshape(_NROWS)
    kvsel = _gather(gidxt, tab).reshape(H, NSEL, N, _RW)

    sout = _s35(q, gidx, kvsel)

    h1 = _s4a(cout, sout, wout, glog, ex, p['Wo'], src2, p['bo'])
    out = _s4b(h1, p)
    return out.reshape(1, N, D)


# speedup vs baseline: 14.6361x; 1.0005x over previous
import functools

import jax
import jax.numpy as jnp
from jax import lax
from jax.experimental import pallas as pl
from jax.experimental.pallas import tpu as pltpu
from jax.experimental.pallas import tpu_sc as plsc

D = 1024
H = 16
DH = 64
DFF = 4096
N = 2048
BC = 4
SC = 2
BS = 4
NSEL = 2
RT = 256
NRT = N // RT
RT3 = 512
RT35 = 512
NC2 = N // SC
NB = N // BS
SCALE = DH ** -0.5

_call = pl.pallas_call


def _s1_body(src_r, wq_r, wkv_r, wg_r, g1_r, b1_r, bg_r,
             q_r, kv_r, g_r):
    x = src_r[...]
    m = jnp.mean(x, -1, keepdims=True)
    va = jnp.mean((x - m) ** 2, -1, keepdims=True)
    xln = (x - m) / jnp.sqrt(va + 1e-5) * g1_r[...] + b1_r[...]
    q_r[...] = xln @ wq_r[...]
    kv_r[...] = xln @ wkv_r[...]
    g_r[...] = xln @ wg_r[...] + bg_r[...]


def _s1(src, wq, wkv, wg_r, ln1g, ln1b, bg_r):
    full = lambda a, b: pl.BlockSpec((a, b), lambda i: (0, 0))
    return _call(
        _s1_body,
        grid=(NRT,),
        in_specs=[
            pl.BlockSpec((RT, D), lambda i: (i, 0)),
            full(D, H * DH), full(D, 2 * H * DH), full(D, H * 3),
            full(1, D), full(1, D), full(1, H * 3),
        ],
        out_specs=[
            pl.BlockSpec((RT, H * DH), lambda i: (i, 0)),
            pl.BlockSpec((RT, 2 * H * DH), lambda i: (i, 0)),
            pl.BlockSpec((RT, H * 3), lambda i: (i, 0)),
        ],
        out_shape=[
            jax.ShapeDtypeStruct((N, H * DH), jnp.float32),
            jax.ShapeDtypeStruct((N, 2 * H * DH), jnp.float32),
            jax.ShapeDtypeStruct((N, H * 3), jnp.float32),
        ],
    )(src, wq, wkv, wg_r, ln1g.reshape(1, D), ln1b.reshape(1, D),
      bg_r.reshape(1, H * 3))


def _s2_body(ev_r, od_r, wkc_r, bkc_r, wvc_r, bvc_r, ck_r, cv_r):
    ev = ev_r[...]
    od = od_r[...]
    kev, vev = ev[:, :DH], ev[:, DH:]
    kod, vod = od[:, :DH], od[:, DH:]
    z = jnp.zeros((1, DH), jnp.float32)
    kev1 = jnp.concatenate([kev[1:], z], axis=0)
    kod1 = jnp.concatenate([kod[1:], z], axis=0)
    vev1 = jnp.concatenate([vev[1:], z], axis=0)
    vod1 = jnp.concatenate([vod[1:], z], axis=0)
    ckc = jnp.concatenate([kev, kod, kev1, kod1], axis=1)
    cvc = jnp.concatenate([vev, vod, vev1, vod1], axis=1)
    ck_r[0] = ckc @ wkc_r[...] + bkc_r[...]
    cv_r[0] = cvc @ wvc_r[...] + bvc_r[...]


def _s2(kv2, p):
    blk = pl.BlockSpec((1, NC2, DH), lambda h: (h, 0, 0))
    full = lambda a, b: pl.BlockSpec((a, b), lambda h: (0, 0))
    return _call(
        _s2_body,
        grid=(H,),
        in_specs=[pl.BlockSpec((NC2, 2 * DH), lambda h: (0, h)),
                  pl.BlockSpec((NC2, 2 * DH), lambda h: (0, H + h)),
                  full(BC * DH, DH), full(1, DH), full(BC * DH, DH), full(1, DH)],
        out_specs=[blk, blk],
        out_shape=[jax.ShapeDtypeStruct((H, NC2, DH), jnp.float32),
                   jax.ShapeDtypeStruct((H, NC2, DH), jnp.float32)],
    )(kv2, kv2, p['Wkc'], p['bkc'].reshape(1, DH),
      p['Wvc'], p['bvc'].reshape(1, DH))


def _s3_body(q_r, kv_r, kvb_r, ck_r, cv_r, cout_r, wout_r, gidx_r):
    h2 = pl.program_id(0)
    rt = pl.program_id(1)
    bprev = jnp.maximum(rt - 1, 0)
    brid = lax.broadcasted_iota(jnp.int32, (N // RT3, 4 * DH), 0)
    brow_all = jnp.sum(jnp.where(brid == bprev, kvb_r[...], 0.0),
                       axis=0, keepdims=True)
    irow = rt * RT3 + lax.broadcasted_iota(jnp.int32, (RT3, NC2), 0)
    col = lax.broadcasted_iota(jnp.int32, (RT3, NC2), 1)
    cmask = (2 * col + BC - 1) <= irow
    scond = (col % 2 == 0) & (2 * col <= irow)
    c2 = lax.broadcasted_iota(jnp.int32, (RT3, NSEL), 1)
    ipc = rt * RT3 + lax.broadcasted_iota(jnp.int32, (RT3, 1), 0)
    valid0 = ipc >= 1
    anyvis = ipc >= (BC - 1)

    for p in range(2):
        q = q_r[:, p * DH:(p + 1) * DH]
        ck = ck_r[p]
        cv = cv_r[p]
        clog = lax.dot_general(q, ck, (((1,), (1,)), ((), ()))) * SCALE
        clogm = jnp.where(cmask, clog, -1e9)
        rowmax = jnp.max(clogm, -1, keepdims=True)
        e = jnp.exp(clogm - rowmax)
        den = jnp.sum(e, -1, keepdims=True)
        cout = lax.dot_general(e, cv, (((1,), (0,)), ((), ())))
        cout_r[p] = jnp.where(anyvis, cout / den, 0.0)

        esh = jnp.concatenate([e[:, 1:], jnp.zeros((RT3, 1), jnp.float32)],
                              axis=1)
        pair = e + esh
        scores = jnp.where(scond, pair, -1.0)
        m1 = jnp.max(scores, -1, keepdims=True)
        i1 = jnp.min(jnp.where(scores == m1, col, NC2 * 4), -1, keepdims=True)
        scores2 = jnp.where(col == i1, -1.0, scores)
        m2 = jnp.max(scores2, -1, keepdims=True)
        i2 = jnp.min(jnp.where(scores2 == m2, col, NC2 * 4), -1, keepdims=True)
        sel1 = i1 // 2
        sel2 = i2 // 2
        gidx_r[p] = jnp.where(c2 == 0, sel1, sel2) + (h2 * 2 + p) * NB

        kvtile = kv_r[:, p * 2 * DH:(p + 1) * 2 * DH]
        brow = brow_all[:, p * 2 * DH:(p + 1) * 2 * DH]
        kvprev = jnp.concatenate([brow, kvtile[:-1]], axis=0)
        k = kvtile[:, :DH]
        v = kvtile[:, DH:]
        kp = kvprev[:, :DH]
        vp = kvprev[:, DH:]
        d1 = jnp.sum(q * k, -1, keepdims=True) * SCALE
        d0 = jnp.sum(q * kp, -1, keepdims=True) * SCALE
        d0m = jnp.where(valid0, d0, -1e9)
        mw = jnp.maximum(d0m, d1)
        e0 = jnp.where(valid0, jnp.exp(d0m - mw), 0.0)
        e1 = jnp.exp(d1 - mw)
        wout_r[p] = (e0 * vp + e1 * v) / (e0 + e1)


def _s3(q, kv, kvb, ck3, cv3):
    hblk = pl.BlockSpec((2, RT3, DH), lambda h2, r: (h2, r, 0))
    return _call(
        _s3_body,
        grid=(H // 2, N // RT3),
        in_specs=[pl.BlockSpec((RT3, 2 * DH), lambda h2, r: (r, h2)),
                  pl.BlockSpec((RT3, 4 * DH), lambda h2, r: (r, h2)),
                  pl.BlockSpec((N // RT3, 4 * DH), lambda h2, r: (0, h2)),
                  pl.BlockSpec((2, NC2, DH), lambda h2, r: (h2, 0, 0)),
                  pl.BlockSpec((2, NC2, DH), lambda h2, r: (h2, 0, 0))],
        out_specs=[hblk, hblk,
                   pl.BlockSpec((2, RT3, NSEL), lambda h2, r: (h2, r, 0))],
        out_shape=[jax.ShapeDtypeStruct((H, N, DH), jnp.float32),
                   jax.ShapeDtypeStruct((H, N, DH), jnp.float32),
                   jax.ShapeDtypeStruct((H, N, NSEL), jnp.int32)],
    )(q, kv, kvb, ck3, cv3)


_NROWS = H * N * NSEL
_RW = BS * 2 * DH
_NW = 32
_CH = 64
_RPW = _NROWS // _NW
_NCHUNK = _RPW // _CH


def _scg_body(gidx_hbm, tab_hbm, out_hbm,
              idx0_v, idx1_v, rows0_v, rows1_v, sem0, sem1):
    wid = lax.axis_index("s") * 2 + lax.axis_index("c")
    base = wid * _RPW

    def body(c2, carry):
        off0 = base + (2 * c2) * _CH
        off1 = off0 + _CH
        pltpu.sync_copy(gidx_hbm.at[pl.ds(off0, _CH)], idx0_v)
        g0 = pltpu.async_copy(tab_hbm.at[idx0_v], rows0_v, sem0)
        pltpu.sync_copy(gidx_hbm.at[pl.ds(off1, _CH)], idx1_v)
        g1 = pltpu.async_copy(tab_hbm.at[idx1_v], rows1_v, sem1)
        g0.wait()
        pltpu.sync_copy(rows0_v, out_hbm.at[pl.ds(off0, _CH)])
        g1.wait()
        pltpu.sync_copy(rows1_v, out_hbm.at[pl.ds(off1, _CH)])
        return carry

    lax.fori_loop(0, _NCHUNK // 2, body, 0)


def _gather(gidxt, tab):
    mesh = plsc.VectorSubcoreMesh(core_axis_name="c", subcore_axis_name="s")
    f = pl.kernel(
        _scg_body,
        mesh=mesh,
        out_type=jax.ShapeDtypeStruct((_NROWS, _RW), jnp.float32),
        scratch_types=[pltpu.VMEM((_CH,), jnp.int32),
                       pltpu.VMEM((_CH,), jnp.int32),
                       pltpu.VMEM((_CH, _RW), jnp.float32),
                       pltpu.VMEM((_CH, _RW), jnp.float32),
                       pltpu.SemaphoreType.DMA,
                       pltpu.SemaphoreType.DMA],
    )
    return f(gidxt, tab)


def _s35_body(q_r, gidx_r, kv_r, sout_r):
    h2 = pl.program_id(0)
    rt = pl.program_id(1)
    ipc = rt * RT35 + lax.broadcasted_iota(jnp.int32, (RT35, 1), 0)
    zpad = jnp.zeros((RT35, DH), jnp.float32)
    for p in range(2):
        q = q_r[:, p * DH:(p + 1) * DH]
        qz = jnp.concatenate([q, zpad], axis=1)
        sel = gidx_r[p] - (h2 * 2 + p) * NB
        logs = []
        toks = []
        for s in range(NSEL):
            sel_s = sel[:, s:s + 1]
            for t in range(BS):
                kv_t = kv_r[p, s, :, t * 2 * DH:(t + 1) * 2 * DH]
                lj = jnp.sum(qz * kv_t, -1, keepdims=True) * SCALE
                tm = (sel_s * BS + t) <= ipc
                logs.append(jnp.where(tm, lj, -1e9))
                toks.append(kv_t)
        m = functools.reduce(jnp.maximum, logs)
        es = [jnp.exp(l - m) for l in logs]
        den = functools.reduce(jnp.add, es)
        acc = es[0] * toks[0]
        for j in range(1, NSEL * BS):
            acc = acc + es[j] * toks[j]
        sout_r[p] = acc[:, DH:] / den


def _s35(q, gidx, kvsel):
    return _call(
        _s35_body,
        grid=(H // 2, N // RT35),
        in_specs=[pl.BlockSpec((RT35, 2 * DH), lambda h2, r: (r, h2)),
                  pl.BlockSpec((2, RT35, NSEL), lambda h2, r: (h2, r, 0)),
                  pl.BlockSpec((2, NSEL, RT35, _RW),
                               lambda h2, r: (h2, 0, r, 0))],
        out_specs=[pl.BlockSpec((2, RT35, DH), lambda h2, r: (h2, r, 0))],
        out_shape=[jax.ShapeDtypeStruct((H, N, DH), jnp.float32)],
    )(q, gidx, kvsel)[0]


def _s4a_body(cout_r, sout_r, wout_r, g_r, ex_r, wo_r, src_r, bo_r, h1_r):
    j = pl.program_id(1)
    gx = jax.nn.sigmoid(g_r[...]) @ ex_r[0]
    cat = lambda x_r: jnp.concatenate([x_r[0], x_r[1], x_r[2], x_r[3]], axis=1)
    o4 = (gx[:, 0:4 * DH] * cat(cout_r)
          + gx[:, 4 * DH:8 * DH] * cat(sout_r)
          + gx[:, 8 * DH:12 * DH] * cat(wout_r))
    part = o4 @ wo_r[...]

    @pl.when(j == 0)
    def _():
        h1_r[...] = src_r[...] + bo_r[...] + part

    @pl.when(j != 0)
    def _():
        h1_r[...] += part


def _s4a(cout, sout, wout, g, ex, wo, src, bo):
    hblk = pl.BlockSpec((4, RT, DH), lambda r, j: (j, r, 0))
    return _call(
        _s4a_body,
        grid=(NRT, 4),
        in_specs=[hblk, hblk, hblk,
                  pl.BlockSpec((RT, H * 3), lambda r, j: (r, 0)),
                  pl.BlockSpec((1, H * 3, 12 * DH), lambda r, j: (j, 0, 0)),
                  pl.BlockSpec((4 * DH, D), lambda r, j: (j, 0)),
                  pl.BlockSpec((RT, D), lambda r, j: (r, 0)),
                  pl.BlockSpec((1, D), lambda r, j: (0, 0))],
        out_specs=[pl.BlockSpec((RT, D), lambda r, j: (r, 0))],
        out_shape=[jax.ShapeDtypeStruct((N, D), jnp.float32)],
    )(cout, sout, wout, g, ex, wo, src, bo.reshape(1, D))[0]


def _s4b_body(h1_r, g2_r, b2ln_r, w1_r, b1_r, w2_r, b2_r, out_r, y_scr):
    j = pl.program_id(1)

    @pl.when(j == 0)
    def _():
        x = h1_r[...]
        m = jnp.mean(x, -1, keepdims=True)
        va = jnp.mean((x - m) ** 2, -1, keepdims=True)
        y_scr[...] = (x - m) / jnp.sqrt(va + 1e-5) * g2_r[...] + b2ln_r[...]
        out_r[...] = x + b2_r[...]

    hmid = jax.nn.gelu(y_scr[...] @ w1_r[...] + b1_r[...])
    out_r[...] += hmid @ w2_r[...]


def _s4b(h1, p):
    JD = DFF // 8
    return _call(
        _s4b_body,
        grid=(NRT, 8),
        in_specs=[pl.BlockSpec((RT, D), lambda r, j: (r, 0)),
                  pl.BlockSpec((1, D), lambda r, j: (0, 0)),
                  pl.BlockSpec((1, D), lambda r, j: (0, 0)),
                  pl.BlockSpec((D, JD), lambda r, j: (0, j)),
                  pl.BlockSpec((1, JD), lambda r, j: (0, j)),
                  pl.BlockSpec((JD, D), lambda r, j: (j, 0)),
                  pl.BlockSpec((1, D), lambda r, j: (0, 0))],
        out_specs=[pl.BlockSpec((RT, D), lambda r, j: (r, 0))],
        out_shape=[jax.ShapeDtypeStruct((N, D), jnp.float32)],
        scratch_shapes=[pltpu.VMEM((RT, D), jnp.float32)],
    )(h1, p['ln2_g'].reshape(1, D), p['ln2_b'].reshape(1, D),
      p['W1'], p['b1'].reshape(1, DFF), p['W2'], p['b2'].reshape(1, D))[0]


def kernel(src, params):
    p = params
    src2 = src[0]

    wkv = jnp.concatenate([p['Wk'].reshape(D, H, DH),
                           p['Wv'].reshape(D, H, DH)], axis=2).reshape(D, 2 * H * DH)
    wg_r = p['Wg'].reshape(D, H, 3).transpose(0, 2, 1).reshape(D, H * 3)
    bg_r = p['bg'].reshape(H, 3).T.reshape(H * 3)
    gidx48 = jnp.arange(H * 3)
    cidx = jnp.arange(12 * DH)
    ex = (gidx48[None, :, None]
          == ((cidx[None, None, :] // (4 * DH)) * H
              + 4 * jnp.arange(4)[:, None, None]
              + (cidx[None, None, :] % (4 * DH)) // DH)).astype(jnp.float32)

    q, kv, glog = _s1(src2, p['Wq'], wkv, wg_r, p['ln1_g'], p['ln1_b'], bg_r)

    kvb = kv[RT3 - 1::RT3]
    kv2 = kv.reshape(NC2, 2 * 2 * H * DH)

    ck3, cv3 = _s2(kv2, p)
    cout, wout, gidx = _s3(q, kv, kvb, ck3, cv3)

    tab = kv.reshape(NB, BS, H, 2 * DH).transpose(2, 0, 1, 3).reshape(H * NB, _RW)
    gidxt = gidx.transpose(0, 2, 1).reshape(_NROWS)
    kvsel = _gather(gidxt, tab).reshape(H, NSEL, N, _RW)

    sout = _s35(q, gidx, kvsel)

    h1 = _s4a(cout, sout, wout, glog, ex, p['Wo'], src2, p['bo'])
    out = _s4b(h1, p)
    return out.reshape(1, N, D)


# speedup vs baseline: 15.2606x; 1.0427x over previous
import functools

import jax
import jax.numpy as jnp
from jax import lax
from jax.experimental import pallas as pl
from jax.experimental.pallas import tpu as pltpu
from jax.experimental.pallas import tpu_sc as plsc

D = 1024
H = 16
DH = 64
DFF = 4096
N = 2048
BC = 4
SC = 2
BS = 4
NSEL = 2
RT = 256
NRT = N // RT
RT3 = 512
RT35 = 512
NC2 = N // SC
NB = N // BS
SCALE = DH ** -0.5

_call = pl.pallas_call


def _s1_body(src_r, wq_r, wkv_r, wg_r, g1_r, b1_r, bg_r,
             q_r, kv_r, g_r):
    x = src_r[...]
    m = jnp.mean(x, -1, keepdims=True)
    va = jnp.mean((x - m) ** 2, -1, keepdims=True)
    xln = (x - m) / jnp.sqrt(va + 1e-5) * g1_r[...] + b1_r[...]
    q_r[...] = xln @ wq_r[...]
    kv_r[...] = xln @ wkv_r[...]
    g_r[...] = xln @ wg_r[...] + bg_r[...]


def _s1(src, wq, wkv, wg_r, ln1g, ln1b, bg_r):
    full = lambda a, b: pl.BlockSpec((a, b), lambda i: (0, 0))
    return _call(
        _s1_body,
        grid=(NRT,),
        in_specs=[
            pl.BlockSpec((RT, D), lambda i: (i, 0)),
            full(D, H * DH), full(D, 2 * H * DH), full(D, H * 3),
            full(1, D), full(1, D), full(1, H * 3),
        ],
        out_specs=[
            pl.BlockSpec((RT, H * DH), lambda i: (i, 0)),
            pl.BlockSpec((RT, 2 * H * DH), lambda i: (i, 0)),
            pl.BlockSpec((RT, H * 3), lambda i: (i, 0)),
        ],
        out_shape=[
            jax.ShapeDtypeStruct((N, H * DH), jnp.float32),
            jax.ShapeDtypeStruct((N, 2 * H * DH), jnp.float32),
            jax.ShapeDtypeStruct((N, H * 3), jnp.float32),
        ],
    )(src, wq, wkv, wg_r, ln1g.reshape(1, D), ln1b.reshape(1, D),
      bg_r.reshape(1, H * 3))


def _s2_body(ev_r, od_r, wkc_r, bkc_r, wvc_r, bvc_r, ck_r, cv_r):
    ev = ev_r[...]
    od = od_r[...]
    kev, vev = ev[:, :DH], ev[:, DH:]
    kod, vod = od[:, :DH], od[:, DH:]
    z = jnp.zeros((1, DH), jnp.float32)
    kev1 = jnp.concatenate([kev[1:], z], axis=0)
    kod1 = jnp.concatenate([kod[1:], z], axis=0)
    vev1 = jnp.concatenate([vev[1:], z], axis=0)
    vod1 = jnp.concatenate([vod[1:], z], axis=0)
    ckc = jnp.concatenate([kev, kod, kev1, kod1], axis=1)
    cvc = jnp.concatenate([vev, vod, vev1, vod1], axis=1)
    ck_r[0] = ckc @ wkc_r[...] + bkc_r[...]
    cv_r[0] = cvc @ wvc_r[...] + bvc_r[...]


def _s2(kv2, p):
    blk = pl.BlockSpec((1, NC2, DH), lambda h: (h, 0, 0))
    full = lambda a, b: pl.BlockSpec((a, b), lambda h: (0, 0))
    return _call(
        _s2_body,
        grid=(H,),
        in_specs=[pl.BlockSpec((NC2, 2 * DH), lambda h: (0, h)),
                  pl.BlockSpec((NC2, 2 * DH), lambda h: (0, H + h)),
                  full(BC * DH, DH), full(1, DH), full(BC * DH, DH), full(1, DH)],
        out_specs=[blk, blk],
        out_shape=[jax.ShapeDtypeStruct((H, NC2, DH), jnp.float32),
                   jax.ShapeDtypeStruct((H, NC2, DH), jnp.float32)],
    )(kv2, kv2, p['Wkc'], p['bkc'].reshape(1, DH),
      p['Wvc'], p['bvc'].reshape(1, DH))


def _s3_body(hoff, q_r, kv_r, kvb_r, ck_r, cv_r, cout_r, wout_r, gidx_r):
    h2 = pl.program_id(0)
    rt = pl.program_id(1)
    bprev = jnp.maximum(rt - 1, 0)
    brid = lax.broadcasted_iota(jnp.int32, (N // RT3, 4 * DH), 0)
    brow_all = jnp.sum(jnp.where(brid == bprev, kvb_r[...], 0.0),
                       axis=0, keepdims=True)
    irow = rt * RT3 + lax.broadcasted_iota(jnp.int32, (RT3, NC2), 0)
    col = lax.broadcasted_iota(jnp.int32, (RT3, NC2), 1)
    cmask = (2 * col + BC - 1) <= irow
    scond = (col % 2 == 0) & (2 * col <= irow)
    c2 = lax.broadcasted_iota(jnp.int32, (RT3, NSEL), 1)
    ipc = rt * RT3 + lax.broadcasted_iota(jnp.int32, (RT3, 1), 0)
    valid0 = ipc >= 1
    anyvis = ipc >= (BC - 1)

    for p in range(2):
        q = q_r[:, p * DH:(p + 1) * DH]
        ck = ck_r[p]
        cv = cv_r[p]
        clog = lax.dot_general(q, ck, (((1,), (1,)), ((), ()))) * SCALE
        clogm = jnp.where(cmask, clog, -1e9)
        rowmax = jnp.max(clogm, -1, keepdims=True)
        e = jnp.exp(clogm - rowmax)
        den = jnp.sum(e, -1, keepdims=True)
        cout = lax.dot_general(e, cv, (((1,), (0,)), ((), ())))
        cout_r[p] = jnp.where(anyvis, cout / den, 0.0)

        esh = jnp.concatenate([e[:, 1:], jnp.zeros((RT3, 1), jnp.float32)],
                              axis=1)
        pair = e + esh
        scores = jnp.where(scond, pair, -1.0)
        m1 = jnp.max(scores, -1, keepdims=True)
        i1 = jnp.min(jnp.where(scores == m1, col, NC2 * 4), -1, keepdims=True)
        scores2 = jnp.where(col == i1, -1.0, scores)
        m2 = jnp.max(scores2, -1, keepdims=True)
        i2 = jnp.min(jnp.where(scores2 == m2, col, NC2 * 4), -1, keepdims=True)
        sel1 = i1 // 2
        sel2 = i2 // 2
        gidx_r[p] = (jnp.where(c2 == 0, sel1, sel2)
                     + (hoff * (H // 2) + h2 * 2 + p) * NB)

        kvtile = kv_r[:, p * 2 * DH:(p + 1) * 2 * DH]
        brow = brow_all[:, p * 2 * DH:(p + 1) * 2 * DH]
        kvprev = jnp.concatenate([brow, kvtile[:-1]], axis=0)
        k = kvtile[:, :DH]
        v = kvtile[:, DH:]
        kp = kvprev[:, :DH]
        vp = kvprev[:, DH:]
        d1 = jnp.sum(q * k, -1, keepdims=True) * SCALE
        d0 = jnp.sum(q * kp, -1, keepdims=True) * SCALE
        d0m = jnp.where(valid0, d0, -1e9)
        mw = jnp.maximum(d0m, d1)
        e0 = jnp.where(valid0, jnp.exp(d0m - mw), 0.0)
        e1 = jnp.exp(d1 - mw)
        wout_r[p] = (e0 * vp + e1 * v) / (e0 + e1)


def _s3(q, kv, kvb, ck3, cv3, hoff):
    HH = H // 2
    ho4 = hoff * (HH // 2)
    hblk = pl.BlockSpec((2, RT3, DH), lambda h2, r: (h2, r, 0))
    return _call(
        functools.partial(_s3_body, hoff),
        grid=(HH // 2, N // RT3),
        in_specs=[pl.BlockSpec((RT3, 2 * DH), lambda h2, r: (r, ho4 + h2)),
                  pl.BlockSpec((RT3, 4 * DH), lambda h2, r: (r, ho4 + h2)),
                  pl.BlockSpec((N // RT3, 4 * DH),
                               lambda h2, r: (0, ho4 + h2)),
                  pl.BlockSpec((2, NC2, DH), lambda h2, r: (ho4 + h2, 0, 0)),
                  pl.BlockSpec((2, NC2, DH), lambda h2, r: (ho4 + h2, 0, 0))],
        out_specs=[hblk, hblk,
                   pl.BlockSpec((2, RT3, NSEL), lambda h2, r: (h2, r, 0))],
        out_shape=[jax.ShapeDtypeStruct((HH, N, DH), jnp.float32),
                   jax.ShapeDtypeStruct((HH, N, DH), jnp.float32),
                   jax.ShapeDtypeStruct((HH, N, NSEL), jnp.int32)],
    )(q, kv, kvb, ck3, cv3)


_NROWS = H * N * NSEL
_RW = BS * 2 * DH
_NW = 32
_CH = 64
_RPW = _NROWS // _NW
_NCHUNK = _RPW // _CH


def _scg_body(rpw, gidx_hbm, tab_hbm, out_hbm,
              idx0_v, idx1_v, rows0_v, rows1_v, sem0, sem1):
    wid = lax.axis_index("s") * 2 + lax.axis_index("c")
    base = wid * rpw

    def body(c2, carry):
        off0 = base + (2 * c2) * _CH
        off1 = off0 + _CH
        pltpu.sync_copy(gidx_hbm.at[pl.ds(off0, _CH)], idx0_v)
        g0 = pltpu.async_copy(tab_hbm.at[idx0_v], rows0_v, sem0)
        pltpu.sync_copy(gidx_hbm.at[pl.ds(off1, _CH)], idx1_v)
        g1 = pltpu.async_copy(tab_hbm.at[idx1_v], rows1_v, sem1)
        g0.wait()
        pltpu.sync_copy(rows0_v, out_hbm.at[pl.ds(off0, _CH)])
        g1.wait()
        pltpu.sync_copy(rows1_v, out_hbm.at[pl.ds(off1, _CH)])
        return carry

    lax.fori_loop(0, rpw // _CH // 2, body, 0)


def _gather(gidxt, tab, nrows):
    mesh = plsc.VectorSubcoreMesh(core_axis_name="c", subcore_axis_name="s")
    f = pl.kernel(
        functools.partial(_scg_body, nrows // _NW),
        mesh=mesh,
        out_type=jax.ShapeDtypeStruct((nrows, _RW), jnp.float32),
        scratch_types=[pltpu.VMEM((_CH,), jnp.int32),
                       pltpu.VMEM((_CH,), jnp.int32),
                       pltpu.VMEM((_CH, _RW), jnp.float32),
                       pltpu.VMEM((_CH, _RW), jnp.float32),
                       pltpu.SemaphoreType.DMA,
                       pltpu.SemaphoreType.DMA],
    )
    return f(gidxt, tab)


def _s35_body(hoff, q_r, gidx_r, kv_r, sout_r):
    h2 = pl.program_id(0)
    rt = pl.program_id(1)
    ipc = rt * RT35 + lax.broadcasted_iota(jnp.int32, (RT35, 1), 0)
    zpad = jnp.zeros((RT35, DH), jnp.float32)
    for p in range(2):
        q = q_r[:, p * DH:(p + 1) * DH]
        qz = jnp.concatenate([q, zpad], axis=1)
        sel = (gidx_r[p]
               - (hoff * (H // 2) + h2 * 2 + p) * NB)
        logs = []
        toks = []
        for s in range(NSEL):
            sel_s = sel[:, s:s + 1]
            for t in range(BS):
                kv_t = kv_r[p, s, :, t * 2 * DH:(t + 1) * 2 * DH]
                lj = jnp.sum(qz * kv_t, -1, keepdims=True) * SCALE
                tm = (sel_s * BS + t) <= ipc
                logs.append(jnp.where(tm, lj, -1e9))
                toks.append(kv_t)
        m = functools.reduce(jnp.maximum, logs)
        es = [jnp.exp(l - m) for l in logs]
        den = functools.reduce(jnp.add, es)
        acc = es[0] * toks[0]
        for j in range(1, NSEL * BS):
            acc = acc + es[j] * toks[j]
        sout_r[p] = acc[:, DH:] / den


def _s35(q, gidx, kvsel, hoff):
    HH = H // 2
    ho4 = hoff * (HH // 2)
    return _call(
        functools.partial(_s35_body, hoff),
        grid=(HH // 2, N // RT35),
        in_specs=[pl.BlockSpec((RT35, 2 * DH), lambda h2, r: (r, ho4 + h2)),
                  pl.BlockSpec((2, RT35, NSEL), lambda h2, r: (h2, r, 0)),
                  pl.BlockSpec((2, NSEL, RT35, _RW),
                               lambda h2, r: (h2, 0, r, 0))],
        out_specs=[pl.BlockSpec((2, RT35, DH), lambda h2, r: (h2, r, 0))],
        out_shape=[jax.ShapeDtypeStruct((HH, N, DH), jnp.float32)],
    )(q, gidx, kvsel)[0]


def _s4a_body(cout_r, sout_r, wout_r, g_r, ex_r, wo_r, src_r, bo_r, h1_r):
    j = pl.program_id(1)
    gx = jax.nn.sigmoid(g_r[...]) @ ex_r[0]
    cat = lambda x_r: jnp.concatenate([x_r[0], x_r[1], x_r[2], x_r[3]], axis=1)
    o4 = (gx[:, 0:4 * DH] * cat(cout_r)
          + gx[:, 4 * DH:8 * DH] * cat(sout_r)
          + gx[:, 8 * DH:12 * DH] * cat(wout_r))
    part = o4 @ wo_r[...]

    @pl.when(j == 0)
    def _():
        h1_r[...] = src_r[...] + bo_r[...] + part

    @pl.when(j != 0)
    def _():
        h1_r[...] += part


def _s4a(cout, sout, wout, g, ex, wo, src, bo):
    hblk = pl.BlockSpec((4, RT, DH), lambda r, j: (j, r, 0))
    return _call(
        _s4a_body,
        grid=(NRT, 4),
        in_specs=[hblk, hblk, hblk,
                  pl.BlockSpec((RT, H * 3), lambda r, j: (r, 0)),
                  pl.BlockSpec((1, H * 3, 12 * DH), lambda r, j: (j, 0, 0)),
                  pl.BlockSpec((4 * DH, D), lambda r, j: (j, 0)),
                  pl.BlockSpec((RT, D), lambda r, j: (r, 0)),
                  pl.BlockSpec((1, D), lambda r, j: (0, 0))],
        out_specs=[pl.BlockSpec((RT, D), lambda r, j: (r, 0))],
        out_shape=[jax.ShapeDtypeStruct((N, D), jnp.float32)],
    )(cout, sout, wout, g, ex, wo, src, bo.reshape(1, D))[0]


def _s4b_body(h1_r, g2_r, b2ln_r, w1_r, b1_r, w2_r, b2_r, out_r, y_scr):
    j = pl.program_id(1)

    @pl.when(j == 0)
    def _():
        x = h1_r[...]
        m = jnp.mean(x, -1, keepdims=True)
        va = jnp.mean((x - m) ** 2, -1, keepdims=True)
        y_scr[...] = (x - m) / jnp.sqrt(va + 1e-5) * g2_r[...] + b2ln_r[...]
        out_r[...] = x + b2_r[...]

    hmid = jax.nn.gelu(y_scr[...] @ w1_r[...] + b1_r[...])
    out_r[...] += hmid @ w2_r[...]


def _s4b(h1, p):
    JD = DFF // 8
    return _call(
        _s4b_body,
        grid=(NRT, 8),
        in_specs=[pl.BlockSpec((RT, D), lambda r, j: (r, 0)),
                  pl.BlockSpec((1, D), lambda r, j: (0, 0)),
                  pl.BlockSpec((1, D), lambda r, j: (0, 0)),
                  pl.BlockSpec((D, JD), lambda r, j: (0, j)),
                  pl.BlockSpec((1, JD), lambda r, j: (0, j)),
                  pl.BlockSpec((JD, D), lambda r, j: (j, 0)),
                  pl.BlockSpec((1, D), lambda r, j: (0, 0))],
        out_specs=[pl.BlockSpec((RT, D), lambda r, j: (r, 0))],
        out_shape=[jax.ShapeDtypeStruct((N, D), jnp.float32)],
        scratch_shapes=[pltpu.VMEM((RT, D), jnp.float32)],
    )(h1, p['ln2_g'].reshape(1, D), p['ln2_b'].reshape(1, D),
      p['W1'], p['b1'].reshape(1, DFF), p['W2'], p['b2'].reshape(1, D))[0]


def kernel(src, params):
    p = params
    src2 = src[0]

    wkv = jnp.concatenate([p['Wk'].reshape(D, H, DH),
                           p['Wv'].reshape(D, H, DH)], axis=2).reshape(D, 2 * H * DH)
    wg_r = p['Wg'].reshape(D, H, 3).transpose(0, 2, 1).reshape(D, H * 3)
    bg_r = p['bg'].reshape(H, 3).T.reshape(H * 3)
    gidx48 = jnp.arange(H * 3)
    cidx = jnp.arange(12 * DH)
    ex = (gidx48[None, :, None]
          == ((cidx[None, None, :] // (4 * DH)) * H
              + 4 * jnp.arange(4)[:, None, None]
              + (cidx[None, None, :] % (4 * DH)) // DH)).astype(jnp.float32)

    q, kv, glog = _s1(src2, p['Wq'], wkv, wg_r, p['ln1_g'], p['ln1_b'], bg_r)

    kvb = kv[RT3 - 1::RT3]
    kv2 = kv.reshape(NC2, 2 * 2 * H * DH)

    ck3, cv3 = _s2(kv2, p)

    tab = kv.reshape(NB, BS, H, 2 * DH).transpose(2, 0, 1, 3).reshape(H * NB, _RW)

    HH = H // 2
    NR2 = _NROWS // 2
    couts, wouts, souts = [], [], []
    gidxs = []
    for hoff in range(2):
        c_, w_, g_ = _s3(q, kv, kvb, ck3, cv3, hoff)
        couts.append(c_)
        wouts.append(w_)
        gidxs.append(g_)
    kvsels = [
        _gather(g_.transpose(0, 2, 1).reshape(NR2), tab, NR2)
        .reshape(HH, NSEL, N, _RW)
        for g_ in gidxs
    ]
    for hoff in range(2):
        souts.append(_s35(q, gidxs[hoff], kvsels[hoff], hoff))

    cout = jnp.concatenate(couts, axis=0)
    wout = jnp.concatenate(wouts, axis=0)
    sout = jnp.concatenate(souts, axis=0)

    h1 = _s4a(cout, sout, wout, glog, ex, p['Wo'], src2, p['bo'])
    out = _s4b(h1, p)
    return out.reshape(1, N, D)
